# baseline (speedup 1.0000x reference)
import functools

import jax
import jax.numpy as jnp
from jax import lax
from jax.experimental import pallas as pl
from jax.experimental.pallas import tpu as pltpu
from jax.experimental.pallas import tpu_sc as plsc

_BATCH = 4096
_SEQ = 200
_D = 64
_TOT = _BATCH * _SEQ
_NC, _NS = 2, 16
_NW = _NC * _NS
_CH = 128
_NCHUNKS = _TOT // _CH
_PER_W = _NCHUNKS // _NW
_NBUF = 3
_BTC = _BATCH // _CH


def _make_gather():
    mesh = plsc.VectorSubcoreMesh(core_axis_name="c", subcore_axis_name="s")

    @functools.partial(
        pl.kernel,
        mesh=mesh,
        compiler_params=pltpu.CompilerParams(
            use_tc_tiling_on_sc=False,
            needs_layout_passes=False,
            disable_bounds_checks=True,
        ),
        out_type=jax.ShapeDtypeStruct((_SEQ * 8, _BTC, 1024), jnp.float32),
        scratch_types=[
            pltpu.VMEM((_PER_W, _CH), jnp.int32),
            pltpu.VMEM((_NBUF, _CH, _D), jnp.float32),
            pltpu.VMEM((_NBUF, 8, 1024), jnp.float32),
            pltpu.SemaphoreType.DMA((_NBUF,)),
            pltpu.SemaphoreType.DMA((_NBUF,)),
        ],
    )
    def gather_kernel(idx_hbm, table_hbm, out_hbm, idx_v, rows_v, tr_v,
                      gsem, ssem):
        wid = lax.axis_index("s") * _NC + lax.axis_index("c")
        ch0 = wid * _PER_W
        pltpu.sync_copy(idx_hbm.at[pl.ds(ch0, _PER_W)], idx_v)

        def start_gather(g, slot):
            pltpu.async_copy(
                table_hbm.at[idx_v.at[g]], rows_v.at[slot], gsem.at[slot]
            )

        def gather_desc(slot):
            return pltpu.make_async_copy(
                table_hbm.at[idx_v.at[0]], rows_v.at[slot], gsem.at[slot]
            )

        def out_slab(g):
            ch = ch0 + g
            s = ch // _BTC
            tc = ch % _BTC
            return out_hbm.at[pl.ds(s * 8, 8), tc]

        def start_store(g, slot):
            pltpu.async_copy(tr_v.at[slot], out_slab(g), ssem.at[slot])

        def store_desc(slot):
            return pltpu.make_async_copy(
                tr_v.at[slot], out_hbm.at[pl.ds(0, 8), 0], ssem.at[slot]
            )

        base = lax.broadcasted_iota(jnp.int32, (16,), 0)
        jvs = [base + 16 * k for k in range(8)]

        def transpose(slot):
            src = rows_v.at[slot]
            dst = tr_v.at[slot]

            def col(c, carry):
                cv = jnp.broadcast_to(c, (16,))
                for k in range(8):
                    v = plsc.load_gather(src, [jvs[k], cv])
                    dst[c >> 3, pl.ds((c & 7) * 128 + 16 * k, 16)] = v
                return carry

            lax.fori_loop(0, _D, col, 0, unroll=8)

        for g in range(_NBUF - 1):
            start_gather(g, g)

        def body(g, carry):
            gp = g + (_NBUF - 1)
            slotp = lax.rem(gp, _NBUF)

            @pl.when(gp < _PER_W)
            def _():
                @pl.when(gp >= _NBUF)
                def _():
                    store_desc(slotp).wait()

                start_gather(gp, slotp)

            slot = lax.rem(g, _NBUF)
            gather_desc(slot).wait()
            transpose(slot)
            start_store(g, slot)
            return carry

        lax.fori_loop(0, _PER_W, body, 0)

        for g in range(_PER_W - _NBUF, _PER_W):
            store_desc(g % _NBUF).wait()

    return gather_kernel


_gather = _make_gather()


def kernel(input_ids, word_embeddings):
    ids = input_ids.T.reshape(_NCHUNKS, _CH).astype(jnp.int32)
    raw = _gather(ids, word_embeddings)
    raw = raw.reshape(_SEQ, 8, _BTC, 8, _CH)
    return raw.transpose(2, 4, 0, 1, 3).reshape(_BATCH, _SEQ, _D)


# speedup vs baseline: 2.0769x; 2.0769x over previous
import functools

import jax
import jax.numpy as jnp
from jax import lax
from jax.experimental import pallas as pl
from jax.experimental.pallas import tpu as pltpu
from jax.experimental.pallas import tpu_sc as plsc

_BATCH = 4096
_SEQ = 200
_D = 64
_TOT = _BATCH * _SEQ
_NC, _NS = 2, 16
_NW = _NC * _NS
_PER_W = _TOT // _NW
_CH = 128
_NCHUNK = _PER_W // _CH
_NBUF = 6


def _make_gather():
    mesh = plsc.VectorSubcoreMesh(core_axis_name="c", subcore_axis_name="s")

    @functools.partial(
        pl.kernel,
        mesh=mesh,
        compiler_params=pltpu.CompilerParams(use_tc_tiling_on_sc=False),
        out_type=jax.ShapeDtypeStruct((_TOT, 2 * _D), jnp.float32),
        scratch_types=[
            pltpu.VMEM((_NCHUNK, _CH), jnp.int32),
            pltpu.VMEM((_NBUF, _CH, _D), jnp.float32),
            pltpu.SemaphoreType.DMA((_NBUF,)),
            pltpu.SemaphoreType.DMA((_NBUF,)),
        ],
    )
    def gather_kernel(idx_hbm, table_hbm, out_hbm, idx_v, rows_v, gsem, ssem):
        wid = lax.axis_index("s") * _NC + lax.axis_index("c")
        pltpu.sync_copy(idx_hbm.at[pl.ds(wid * _NCHUNK, _NCHUNK)], idx_v)
        out_base = wid * _PER_W

        def start_gather(g, slot):
            pltpu.async_copy(
                table_hbm.at[idx_v.at[g]], rows_v.at[slot], gsem.at[slot]
            )

        def gather_desc(slot):
            return pltpu.make_async_copy(
                table_hbm.at[idx_v.at[0]], rows_v.at[slot], gsem.at[slot]
            )

        def start_store(h, slot):
            pltpu.async_copy(
                rows_v.at[slot],
                out_hbm.at[pl.ds(out_base + h * _CH, _CH), pl.ds(0, _D)],
                ssem.at[slot],
            )

        def store_desc(slot):
            return pltpu.make_async_copy(
                rows_v.at[slot],
                out_hbm.at[pl.ds(out_base, _CH), pl.ds(0, _D)],
                ssem.at[slot],
            )

        _LAG = _NBUF - 1

        def body(g, carry):
            slot = lax.rem(g, _NBUF)

            @pl.when(g >= _NBUF)
            def _():
                store_desc(slot).wait()

            start_gather(g, slot)

            @pl.when(g >= _LAG)
            def _():
                h = g - _LAG
                hslot = lax.rem(h, _NBUF)
                gather_desc(hslot).wait()
                start_store(h, hslot)

            return carry

        lax.fori_loop(0, _NCHUNK, body, 0)

        for h in range(_NCHUNK - _LAG, _NCHUNK):
            slot = h % _NBUF
            gather_desc(slot).wait()
            start_store(h, slot)
        for h in range(_NCHUNK - _NBUF, _NCHUNK):
            store_desc(h % _NBUF).wait()

    return gather_kernel


_gather = _make_gather()


def kernel(input_ids, word_embeddings):
    ids = input_ids.reshape(_TOT // _CH, _CH).astype(jnp.int32)
    padded = _gather(ids, word_embeddings)
    return padded[:, :_D].reshape(_BATCH, _SEQ, _D)
